# baseline (speedup 1.0000x reference)
import functools

import jax
import jax.numpy as jnp
from jax import lax
from jax.experimental import pallas as pl
from jax.experimental.pallas import tpu as pltpu
from jax.experimental.pallas import tpu_sc as plsc

N = 64
E = N * (N - 1)
EA_W = 11
EP_W = 3
NPLANE = EA_W + EP_W

_A1 = 0.99535791
_A3 = -0.28868991
_A5 = 0.07933871
_PI = 3.14159265358979323846
_HALF_PI = _PI / 2.0


def _rsqrt16(a, newton=2):
    xi = lax.bitcast_convert_type(a, jnp.int32)
    yi = jnp.int32(0x5F3759DF) - (xi >> 1)
    y = lax.bitcast_convert_type(yi, jnp.float32)
    for _ in range(newton):
        y = y * (1.5 - 0.5 * a * y * y)
    return y


def _atan2_16(y, x):
    ax = jnp.abs(x)
    ay = jnp.abs(y)
    mx = jnp.maximum(ax, ay)
    mn = jnp.minimum(ax, ay)
    t = mn / jnp.maximum(mx, 1e-37)
    t2 = t * t
    p = t * (_A1 + t2 * (_A3 + t2 * _A5))
    p = jnp.where(ay > ax, _HALF_PI - p, p)
    p = jnp.where(x < 0.0, _PI - p, p)
    return jnp.where(y < 0.0, -p, p)


def _make_sc_call(batch):
    info = plsc.get_sparse_core_info()
    nw = info.num_cores * info.num_subcores
    n_echunk = 4
    n_btile = nw // n_echunk
    bt_w = batch // n_btile
    ec_e = E // n_echunk
    ec_tiles = ec_e // 8
    mesh = plsc.VectorSubcoreMesh(core_axis_name="c", subcore_axis_name="s")

    @functools.partial(
        pl.kernel,
        out_type=(
            jax.ShapeDtypeStruct((N, 4 * batch), jnp.float32),
            jax.ShapeDtypeStruct((N, 4 * batch), jnp.float32),
            jax.ShapeDtypeStruct((EA_W, E, batch), jnp.float32),
            jax.ShapeDtypeStruct((EP_W, E, batch), jnp.float32),
        ),
        mesh=mesh,
        compiler_params=pltpu.CompilerParams(
            needs_layout_passes=False, use_tc_tiling_on_sc=True),
        scratch_types=(
            pltpu.VMEM((bt_w, N * 4 // 128 * 128), jnp.float32),
            pltpu.VMEM((N * 128,), jnp.float32),
            pltpu.VMEM((N * 128,), jnp.float32),
            pltpu.VMEM((N * 128,), jnp.float32),
            pltpu.VMEM((N * 128,), jnp.float32),
            pltpu.VMEM((N * 128,), jnp.float32),
            pltpu.VMEM((N * 128,), jnp.float32),
            pltpu.VMEM((N * 128,), jnp.float32),
            pltpu.VMEM((2, EA_W, 8, 128), jnp.float32),
            pltpu.VMEM((8, 128), jnp.float32),
            pltpu.SemaphoreType.DMA,
            pltpu.SemaphoreType.DMA,
        ),
    )
    def sc_call(x_hbm, rel_hbm, rinv_hbm, ea_hbm, ep_hbm,
                x_t, px_t, py_t, vx_t, vy_t, c_t, s_t, spd_t,
                ebuf, stile, sem0, sem1):
        wid = lax.axis_index("s") * info.num_cores + lax.axis_index("c")
        ec = wid // n_btile
        tb = wid % n_btile
        b0 = tb * bt_w
        iota = lax.iota(jnp.int32, 16)
        zv = jnp.zeros((16,), jnp.float32)

        pltpu.sync_copy(x_hbm.at[pl.ds(b0, bt_w), :], x_t)

        @plsc.parallel_loop(0, N * 8, unroll=2)
        def node_body(idx):
            n = idx >> 3
            g = idx & 7
            bv = iota + g * 16
            colv = iota * 0 + n * 4
            px = plsc.load_gather(x_t, [bv, colv])
            py = plsc.load_gather(x_t, [bv, colv + 1])
            vx = plsc.load_gather(x_t, [bv, colv + 2])
            vy = plsc.load_gather(x_t, [bv, colv + 3])
            n2 = vx * vx + vy * vy
            inv = _rsqrt16(jnp.maximum(n2, 1e-30))
            zero = n2 <= 0.0
            c = jnp.where(zero, 1.0, vx * inv)
            s = jnp.where(zero, 0.0, vy * inv)
            spd = n2 * inv
            off = n * 128 + g * 16
            px_t[pl.ds(off, 16)] = px
            py_t[pl.ds(off, 16)] = py
            vx_t[pl.ds(off, 16)] = vx
            vy_t[pl.ds(off, 16)] = vy
            c_t[pl.ds(off, 16)] = c
            s_t[pl.ds(off, 16)] = s
            spd_t[pl.ds(off, 16)] = spd

        for r in range(8):
            for g in range(8):
                stile[r, pl.ds(g * 16, 16)] = zv
        for k in (0, 1, 3):
            def zero_tile(rt, carry, k=k):
                pltpu.sync_copy(
                    stile,
                    rel_hbm.at[pl.ds(rt * 8, 8),
                               pl.ds(k * batch + b0, 128)])
                return carry

            lax.fori_loop(2 * ec, 2 * ec + 2, zero_tile, 0)

        def spd_tile(rt, carry):
            for r in range(8):
                off = rt * 1024 + r * 128
                for g in range(8):
                    stile[r, pl.ds(g * 16, 16)] = (
                        spd_t[pl.ds(off + g * 16, 16)])
            pltpu.sync_copy(
                stile,
                rel_hbm.at[pl.ds(rt * 8, 8), pl.ds(2 * batch + b0, 128)])
            return carry

        lax.fori_loop(2 * ec, 2 * ec + 2, spd_tile, 0)

        for k, src_t, neg in ((0, c_t, False), (1, s_t, True),
                              (2, s_t, False), (3, c_t, False)):
            def rinv_tile(rt, carry, src_t=src_t, neg=neg, k=k):
                for r in range(8):
                    off = rt * 1024 + r * 128
                    for g in range(8):
                        val = src_t[pl.ds(off + g * 16, 16)]
                        stile[r, pl.ds(g * 16, 16)] = -val if neg else val
                pltpu.sync_copy(
                    stile,
                    rinv_hbm.at[pl.ds(rt * 8, 8),
                                pl.ds(k * batch + b0, 128)])
                return carry

            lax.fori_loop(2 * ec, 2 * ec + 2, rinv_tile, 0)

        for p in range(2):
            for k in (7, 8, 10):
                for r in range(8):
                    for g in range(8):
                        ebuf[p, k, r, pl.ds(g * 16, 16)] = zv

        e_base = ec * ec_e
        sems = (sem0, sem1)

        def dma_pairs(p, e_start):
            return (
                (ebuf.at[p],
                 ea_hbm.at[pl.ds(0, EA_W), pl.ds(e_start, 8),
                           pl.ds(b0, 128)]),
                (ebuf.at[p, pl.ds(2, EP_W)],
                 ep_hbm.at[pl.ds(0, EP_W), pl.ds(e_start, 8),
                           pl.ds(b0, 128)]),
            )

        def pair_body(tt, carry):
            for p in range(2):
                et = tt * 2 + p
                e_start = e_base + et * 8

                @pl.when(tt >= 1)
                def _():
                    for src_pl, dst in dma_pairs(p, e_start):
                        pltpu.make_async_copy(src_pl, dst, sems[p]).wait()

                @plsc.parallel_loop(0, 64, unroll=3)
                def edge_body(idx):
                    r = idx >> 3
                    g = idx & 7
                    e = e_start + r
                    i = (e * 16645) >> 20
                    jj = e - i * 63
                    j = jj + (jj >= i).astype(jnp.int32)
                    go = g * 16
                    io = i * 128 + go
                    jo = j * 128 + go
                    px_s = px_t[pl.ds(io, 16)]
                    py_s = py_t[pl.ds(io, 16)]
                    vx_s = vx_t[pl.ds(io, 16)]
                    vy_s = vy_t[pl.ds(io, 16)]
                    px_r = px_t[pl.ds(jo, 16)]
                    py_r = py_t[pl.ds(jo, 16)]
                    cr = c_t[pl.ds(jo, 16)]
                    sr = s_t[pl.ds(jo, 16)]
                    spdr = spd_t[pl.ds(jo, 16)]
                    dx = px_s - px_r
                    dy = py_s - py_r
                    rx = cr * dx + sr * dy
                    ry = cr * dy - sr * dx
                    dot = vx_s * cr + vy_s * sr
                    cross = vy_s * cr - vx_s * sr
                    dtheta = _atan2_16(cross, dot)
                    r2 = rx * rx + ry * ry + 1e-12
                    rad = r2 * _rsqrt16(r2, newton=1)
                    phi = _atan2_16(ry, rx)
                    vals = ((0, rx), (1, ry), (2, dtheta), (3, rad),
                            (4, phi), (5, dot), (6, cross), (9, spdr))
                    for k, v in vals:
                        ebuf[p, k, r, pl.ds(go, 16)] = v

                for src_pl, dst in dma_pairs(p, e_start):
                    pltpu.async_copy(src_pl, dst, sems[p])
            return carry

        lax.fori_loop(0, ec_tiles // 2, pair_body, 0)

        for p in range(2):
            e_start = e_base + (ec_tiles - 2 + p) * 8
            for src_pl, dst in dma_pairs(p, e_start):
                pltpu.make_async_copy(src_pl, dst, sems[p]).wait()

    return sc_call


def kernel(x):
    batch = x.shape[0]
    rel2, rinv2, ea_t, ep_t = _make_sc_call(batch)(x.reshape(batch, N * 4))
    rel_feat = jnp.transpose(rel2.reshape(N, 4, batch), (2, 0, 1))
    rinv = jnp.transpose(rinv2.reshape(N, 4, batch), (2, 0, 1))
    return (
        rel_feat,
        rinv.reshape(batch, N, 2, 2),
        jnp.transpose(ea_t, (2, 1, 0)),
        jnp.transpose(ep_t, (2, 1, 0)),
    )


# speedup vs baseline: 1.0622x; 1.0622x over previous
import functools

import jax
import jax.numpy as jnp
from jax import lax
from jax.experimental import pallas as pl
from jax.experimental.pallas import tpu as pltpu
from jax.experimental.pallas import tpu_sc as plsc

N = 64
E = N * (N - 1)
EA_W = 11
EP_W = 3
NPLANE = EA_W + EP_W

_A1 = 0.97239379
_A3 = -0.19194741
_PI = 3.14159265358979323846
_HALF_PI = _PI / 2.0


def _rsqrt16(a, newton=2):
    xi = lax.bitcast_convert_type(a, jnp.int32)
    yi = jnp.int32(0x5F3759DF) - (xi >> 1)
    y = lax.bitcast_convert_type(yi, jnp.float32)
    for _ in range(newton):
        y = y * (1.5 - 0.5 * a * y * y)
    return y


def _atan2_16(y, x):
    ax = jnp.abs(x)
    ay = jnp.abs(y)
    mx = jnp.maximum(ax, ay)
    mn = jnp.minimum(ax, ay)
    t = mn / jnp.maximum(mx, 1e-37)
    t2 = t * t
    p = t * (_A1 + t2 * _A3)
    p = jnp.where(ay > ax, _HALF_PI - p, p)
    p = jnp.where(x < 0.0, _PI - p, p)
    return jnp.where(y < 0.0, -p, p)


def _make_sc_call(batch):
    info = plsc.get_sparse_core_info()
    nw = info.num_cores * info.num_subcores
    n_echunk = 4
    n_btile = nw // n_echunk
    bt_w = batch // n_btile
    ec_e = E // n_echunk
    ec_tiles = ec_e // 8
    mesh = plsc.VectorSubcoreMesh(core_axis_name="c", subcore_axis_name="s")

    @functools.partial(
        pl.kernel,
        out_type=(
            jax.ShapeDtypeStruct((N, 4 * batch), jnp.float32),
            jax.ShapeDtypeStruct((N, 4 * batch), jnp.float32),
            jax.ShapeDtypeStruct((EA_W, E, batch), jnp.float32),
            jax.ShapeDtypeStruct((EP_W, E, batch), jnp.float32),
        ),
        mesh=mesh,
        compiler_params=pltpu.CompilerParams(
            needs_layout_passes=False, use_tc_tiling_on_sc=True),
        scratch_types=(
            pltpu.VMEM((bt_w, N * 4 // 128 * 128), jnp.float32),
            pltpu.VMEM((N * 128,), jnp.float32),
            pltpu.VMEM((N * 128,), jnp.float32),
            pltpu.VMEM((N * 128,), jnp.float32),
            pltpu.VMEM((N * 128,), jnp.float32),
            pltpu.VMEM((N * 128,), jnp.float32),
            pltpu.VMEM((N * 128,), jnp.float32),
            pltpu.VMEM((N * 128,), jnp.float32),
            pltpu.VMEM((2, EA_W, 8, 128), jnp.float32),
            pltpu.VMEM((8, 128), jnp.float32),
            pltpu.SemaphoreType.DMA,
            pltpu.SemaphoreType.DMA,
        ),
    )
    def sc_call(x_hbm, rel_hbm, rinv_hbm, ea_hbm, ep_hbm,
                x_t, px_t, py_t, vx_t, vy_t, c_t, s_t, spd_t,
                ebuf, stile, sem0, sem1):
        wid = lax.axis_index("s") * info.num_cores + lax.axis_index("c")
        ec = wid // n_btile
        tb = wid % n_btile
        b0 = tb * bt_w
        iota = lax.iota(jnp.int32, 16)
        zv = jnp.zeros((16,), jnp.float32)

        pltpu.sync_copy(x_hbm.at[pl.ds(b0, bt_w), :], x_t)

        @plsc.parallel_loop(0, N * 8, unroll=2)
        def node_body(idx):
            n = idx >> 3
            g = idx & 7
            bv = iota + g * 16
            colv = iota * 0 + n * 4
            px = plsc.load_gather(x_t, [bv, colv])
            py = plsc.load_gather(x_t, [bv, colv + 1])
            vx = plsc.load_gather(x_t, [bv, colv + 2])
            vy = plsc.load_gather(x_t, [bv, colv + 3])
            n2 = vx * vx + vy * vy
            inv = _rsqrt16(jnp.maximum(n2, 1e-30))
            zero = n2 <= 0.0
            c = jnp.where(zero, 1.0, vx * inv)
            s = jnp.where(zero, 0.0, vy * inv)
            spd = n2 * inv
            off = n * 128 + g * 16
            px_t[pl.ds(off, 16)] = px
            py_t[pl.ds(off, 16)] = py
            vx_t[pl.ds(off, 16)] = vx
            vy_t[pl.ds(off, 16)] = vy
            c_t[pl.ds(off, 16)] = c
            s_t[pl.ds(off, 16)] = s
            spd_t[pl.ds(off, 16)] = spd

        for r in range(8):
            for g in range(8):
                stile[r, pl.ds(g * 16, 16)] = zv
        for k in (0, 1, 3):
            def zero_tile(rt, carry, k=k):
                pltpu.sync_copy(
                    stile,
                    rel_hbm.at[pl.ds(rt * 8, 8),
                               pl.ds(k * batch + b0, 128)])
                return carry

            lax.fori_loop(2 * ec, 2 * ec + 2, zero_tile, 0)

        def spd_tile(rt, carry):
            for r in range(8):
                off = rt * 1024 + r * 128
                for g in range(8):
                    stile[r, pl.ds(g * 16, 16)] = (
                        spd_t[pl.ds(off + g * 16, 16)])
            pltpu.sync_copy(
                stile,
                rel_hbm.at[pl.ds(rt * 8, 8), pl.ds(2 * batch + b0, 128)])
            return carry

        lax.fori_loop(2 * ec, 2 * ec + 2, spd_tile, 0)

        for k, src_t, neg in ((0, c_t, False), (1, s_t, True),
                              (2, s_t, False), (3, c_t, False)):
            def rinv_tile(rt, carry, src_t=src_t, neg=neg, k=k):
                for r in range(8):
                    off = rt * 1024 + r * 128
                    for g in range(8):
                        val = src_t[pl.ds(off + g * 16, 16)]
                        stile[r, pl.ds(g * 16, 16)] = -val if neg else val
                pltpu.sync_copy(
                    stile,
                    rinv_hbm.at[pl.ds(rt * 8, 8),
                                pl.ds(k * batch + b0, 128)])
                return carry

            lax.fori_loop(2 * ec, 2 * ec + 2, rinv_tile, 0)

        for p in range(2):
            for k in (7, 8, 10):
                for r in range(8):
                    for g in range(8):
                        ebuf[p, k, r, pl.ds(g * 16, 16)] = zv

        e_base = ec * ec_e
        sems = (sem0, sem1)

        def dma_pairs(p, e_start):
            return (
                (ebuf.at[p],
                 ea_hbm.at[pl.ds(0, EA_W), pl.ds(e_start, 8),
                           pl.ds(b0, 128)]),
                (ebuf.at[p, pl.ds(2, EP_W)],
                 ep_hbm.at[pl.ds(0, EP_W), pl.ds(e_start, 8),
                           pl.ds(b0, 128)]),
            )

        def pair_body(tt, carry):
            for p in range(2):
                et = tt * 2 + p
                e_start = e_base + et * 8

                @pl.when(tt >= 1)
                def _():
                    for src_pl, dst in dma_pairs(p, e_start):
                        pltpu.make_async_copy(src_pl, dst, sems[p]).wait()

                @plsc.parallel_loop(0, 64, unroll=2)
                def edge_body(idx):
                    r = idx >> 3
                    g = idx & 7
                    e = e_start + r
                    i = (e * 16645) >> 20
                    jj = e - i * 63
                    j = jj + (jj >= i).astype(jnp.int32)
                    go = g * 16
                    io = i * 128 + go
                    jo = j * 128 + go
                    px_s = px_t[pl.ds(io, 16)]
                    py_s = py_t[pl.ds(io, 16)]
                    vx_s = vx_t[pl.ds(io, 16)]
                    vy_s = vy_t[pl.ds(io, 16)]
                    px_r = px_t[pl.ds(jo, 16)]
                    py_r = py_t[pl.ds(jo, 16)]
                    cr = c_t[pl.ds(jo, 16)]
                    sr = s_t[pl.ds(jo, 16)]
                    spdr = spd_t[pl.ds(jo, 16)]
                    dx = px_s - px_r
                    dy = py_s - py_r
                    rx = cr * dx + sr * dy
                    ry = cr * dy - sr * dx
                    dot = vx_s * cr + vy_s * sr
                    cross = vy_s * cr - vx_s * sr
                    dtheta = _atan2_16(cross, dot)
                    r2 = rx * rx + ry * ry + 1e-12
                    rad = r2 * _rsqrt16(r2, newton=1)
                    phi = _atan2_16(ry, rx)
                    vals = ((0, rx), (1, ry), (2, dtheta), (3, rad),
                            (4, phi), (5, dot), (6, cross), (9, spdr))
                    for k, v in vals:
                        ebuf[p, k, r, pl.ds(go, 16)] = v

                for src_pl, dst in dma_pairs(p, e_start):
                    pltpu.async_copy(src_pl, dst, sems[p])
            return carry

        lax.fori_loop(0, ec_tiles // 2, pair_body, 0)

        for p in range(2):
            e_start = e_base + (ec_tiles - 2 + p) * 8
            for src_pl, dst in dma_pairs(p, e_start):
                pltpu.make_async_copy(src_pl, dst, sems[p]).wait()

    return sc_call


def kernel(x):
    batch = x.shape[0]
    rel2, rinv2, ea_t, ep_t = _make_sc_call(batch)(x.reshape(batch, N * 4))
    rel_feat = jnp.transpose(rel2.reshape(N, 4, batch), (2, 0, 1))
    rinv = jnp.transpose(rinv2.reshape(N, 4, batch), (2, 0, 1))
    return (
        rel_feat,
        rinv.reshape(batch, N, 2, 2),
        jnp.transpose(ea_t, (2, 1, 0)),
        jnp.transpose(ep_t, (2, 1, 0)),
    )
